# baseline (speedup 1.0000x reference)
import functools

import jax
import jax.numpy as jnp
from jax import lax
from jax.experimental import pallas as pl
from jax.experimental.pallas import tpu as pltpu
from jax.experimental.pallas import tpu_sc as plsc

VOCAB = 1000000
EMB = 64
BATCH = 4096
SEQ = 200

NUM_CORES = 2
NUM_SUBCORES = 16
NW = NUM_CORES * NUM_SUBCORES
SENT_PER_W = BATCH // NW
LANES = 16
NB = SENT_PER_W // LANES

TC_BLK = 32768
MAIN_V = (VOCAB // TC_BLK) * TC_BLK
TAIL_V = VOCAB - MAIN_V


def _matvec_main_body(v_ref, tabt_ref, out_ref):
    out_ref[...] = jax.lax.dot_general(
        v_ref[...],
        tabt_ref[...],
        (((1,), (0,)), ((), ())),
        preferred_element_type=jnp.float32,
    )[0]


def _matvec_tail_body(v_ref, tab_ref, out_ref):
    out_ref[...] = jax.lax.dot_general(
        tab_ref[...],
        v_ref[...],
        (((1,), (1,)), ((), ())),
        preferred_element_type=jnp.float32,
    )[:, 0]


def _tc_matvec(table, v1):
    tabt = table.T
    main = pl.pallas_call(
        _matvec_main_body,
        grid=(MAIN_V // TC_BLK,),
        in_specs=[
            pl.BlockSpec((1, EMB), lambda i: (0, 0)),
            pl.BlockSpec((EMB, TC_BLK), lambda i: (0, i)),
        ],
        out_specs=pl.BlockSpec((TC_BLK,), lambda i: (i,)),
        out_shape=jax.ShapeDtypeStruct((MAIN_V,), jnp.float32),
    )(v1, tabt)
    tail_rows = lax.slice(table, (MAIN_V, 0), (VOCAB, EMB))
    tail = pl.pallas_call(
        _matvec_tail_body,
        in_specs=[
            pl.BlockSpec((1, EMB), lambda: (0, 0)),
            pl.BlockSpec((TAIL_V, EMB), lambda: (0, 0)),
        ],
        out_specs=pl.BlockSpec((TAIL_V,), lambda: (0,)),
        out_shape=jax.ShapeDtypeStruct((TAIL_V,), jnp.float32),
    )(v1, tail_rows)
    return jnp.concatenate([main, tail])


def _sc_body(sent_hbm, tablev_hbm, bias_hbm, out_hbm, idx_v, g_v, bias_v, o_v, sem):
    c = lax.axis_index("c")
    s = lax.axis_index("s")
    wid = s * NUM_CORES + c

    pltpu.sync_copy(bias_hbm, bias_v)
    pltpu.sync_copy(sent_hbm.at[wid], idx_v)
    pltpu.async_copy(tablev_hbm.at[idx_v], g_v, sem).wait()

    bias = bias_v[...]
    inv_l = jnp.float32(1.0 / SEQ)

    def body(r, accs):
        return tuple(
            a + g_v[pl.ds(r * SENT_PER_W + jb * LANES, LANES)]
            for jb, a in enumerate(accs)
        )

    accs = lax.fori_loop(
        0, SEQ, body, tuple(jnp.zeros((LANES,), jnp.float32) for _ in range(NB))
    )

    for jb in range(NB):
        d = accs[jb] * inv_l + bias
        p0 = 1.0 / (1.0 + jnp.exp(-d))
        o_v[0, pl.ds(jb * LANES, LANES)] = p0
        o_v[1, pl.ds(jb * LANES, LANES)] = 1.0 - p0

    pltpu.sync_copy(o_v.at[0], out_hbm.at[0, wid])
    pltpu.sync_copy(o_v.at[1], out_hbm.at[1, wid])


_sc_kernel = functools.partial(
    pl.kernel,
    out_type=jax.ShapeDtypeStruct((2, NW, SENT_PER_W), jnp.float32),
    mesh=plsc.VectorSubcoreMesh(
        core_axis_name="c",
        subcore_axis_name="s",
        num_cores=NUM_CORES,
        num_subcores=NUM_SUBCORES,
    ),
    scratch_types=[
        pltpu.VMEM((SEQ * SENT_PER_W,), jnp.int32),
        pltpu.VMEM((SEQ * SENT_PER_W,), jnp.float32),
        pltpu.VMEM((LANES,), jnp.float32),
        pltpu.VMEM((2, SENT_PER_W), jnp.float32),
        pltpu.SemaphoreType.DMA,
    ],
)(_sc_body)


def kernel(sentence, table, W2, b2):
    v1 = (W2[:, 0] - W2[:, 1]).reshape(1, EMB)
    db = b2[0] - b2[1]
    tablev = _tc_matvec(table, v1)
    sent_t = sentence.reshape(NW, SENT_PER_W, SEQ).transpose(0, 2, 1).reshape(
        NW, SEQ * SENT_PER_W
    )
    bias_arr = jnp.full((LANES,), db, jnp.float32)
    out2 = _sc_kernel(sent_t, tablev, bias_arr)
    return out2.reshape(2, BATCH).T


# speedup vs baseline: 1.1800x; 1.1800x over previous
import functools

import jax
import jax.numpy as jnp
from jax import lax
from jax.experimental import pallas as pl
from jax.experimental.pallas import tpu as pltpu
from jax.experimental.pallas import tpu_sc as plsc

VOCAB = 1000000
EMB = 64
BATCH = 4096
SEQ = 200

NUM_CORES = 2
NUM_SUBCORES = 16
NW = NUM_CORES * NUM_SUBCORES
SENT_PER_W = BATCH // NW
LANES = 16
NB = SENT_PER_W // LANES

TC_BLK = 16384
MAIN_V = (VOCAB // TC_BLK) * TC_BLK
TAIL_V = VOCAB - MAIN_V


def _matvec_main_body(v_ref, tabt_ref, out_ref):
    out_ref[...] = jax.lax.dot_general(
        v_ref[...],
        tabt_ref[...],
        (((1,), (0,)), ((), ())),
        preferred_element_type=jnp.float32,
    )[0]


def _matvec_tail_body(v_ref, tab_ref, out_ref):
    out_ref[...] = jax.lax.dot_general(
        tab_ref[...],
        v_ref[...],
        (((1,), (1,)), ((), ())),
        preferred_element_type=jnp.float32,
    )[:, 0]


def _tc_matvec(table, v1):
    tabt = table.T
    main = pl.pallas_call(
        _matvec_main_body,
        grid=(MAIN_V // TC_BLK,),
        in_specs=[
            pl.BlockSpec((1, EMB), lambda i: (0, 0)),
            pl.BlockSpec((EMB, TC_BLK), lambda i: (0, i)),
        ],
        out_specs=pl.BlockSpec((TC_BLK,), lambda i: (i,)),
        out_shape=jax.ShapeDtypeStruct((MAIN_V,), jnp.float32),
    )(v1, tabt)
    tail_rows = lax.slice(table, (MAIN_V, 0), (VOCAB, EMB))
    tail = pl.pallas_call(
        _matvec_tail_body,
        in_specs=[
            pl.BlockSpec((1, EMB), lambda: (0, 0)),
            pl.BlockSpec((TAIL_V, EMB), lambda: (0, 0)),
        ],
        out_specs=pl.BlockSpec((TAIL_V,), lambda: (0,)),
        out_shape=jax.ShapeDtypeStruct((TAIL_V,), jnp.float32),
    )(v1, tail_rows)
    return jnp.concatenate([main, tail])


def _sc_body(
    sent_hbm, tablev_hbm, bias_hbm, out_hbm, idx_v, g_v, bias_v, o_v, shared_v, sem
):
    c = lax.axis_index("c")
    s = lax.axis_index("s")
    wid = s * NUM_CORES + c

    pltpu.sync_copy(bias_hbm, bias_v)
    pltpu.sync_copy(sent_hbm.at[wid], idx_v)
    @pl.when(s == 0)
    def _stage():
        pltpu.sync_copy(tablev_hbm, shared_v)

    plsc.subcore_barrier()
    pltpu.async_copy(shared_v.at[idx_v], g_v, sem).wait()

    bias = bias_v[...]
    inv_l = jnp.float32(1.0 / SEQ)

    def body(r, accs):
        return tuple(
            a + g_v[pl.ds(r * SENT_PER_W + jb * LANES, LANES)]
            for jb, a in enumerate(accs)
        )

    accs = lax.fori_loop(
        0, SEQ, body, tuple(jnp.zeros((LANES,), jnp.float32) for _ in range(NB))
    )

    for jb in range(NB):
        d = accs[jb] * inv_l + bias
        p0 = 1.0 / (1.0 + jnp.exp(-d))
        o_v[0, pl.ds(jb * LANES, LANES)] = p0
        o_v[1, pl.ds(jb * LANES, LANES)] = 1.0 - p0

    pltpu.sync_copy(o_v.at[0], out_hbm.at[0, wid])
    pltpu.sync_copy(o_v.at[1], out_hbm.at[1, wid])


_sc_kernel = functools.partial(
    pl.kernel,
    out_type=jax.ShapeDtypeStruct((2, NW, SENT_PER_W), jnp.float32),
    mesh=plsc.VectorSubcoreMesh(
        core_axis_name="c",
        subcore_axis_name="s",
        num_cores=NUM_CORES,
        num_subcores=NUM_SUBCORES,
    ),
    scratch_types=[
        pltpu.VMEM((SEQ * SENT_PER_W,), jnp.int32),
        pltpu.VMEM((SEQ * SENT_PER_W,), jnp.float32),
        pltpu.VMEM((LANES,), jnp.float32),
        pltpu.VMEM((2, SENT_PER_W), jnp.float32),
        pltpu.VMEM_SHARED((VOCAB,), jnp.float32),
        pltpu.SemaphoreType.DMA,
    ],
)(_sc_body)


def kernel(sentence, table, W2, b2):
    v1 = (W2[:, 0] - W2[:, 1]).reshape(1, EMB)
    db = b2[0] - b2[1]
    tablev = _tc_matvec(table, v1)
    sent_t = sentence.reshape(NW, SENT_PER_W, SEQ).transpose(0, 2, 1).reshape(
        NW, SEQ * SENT_PER_W
    )
    bias_arr = jnp.full((LANES,), db, jnp.float32)
    out2 = _sc_kernel(sent_t, tablev, bias_arr)
    return out2.reshape(2, BATCH).T


# speedup vs baseline: 1.3191x; 1.1179x over previous
import functools

import jax
import jax.numpy as jnp
from jax import lax
from jax.experimental import pallas as pl
from jax.experimental.pallas import tpu as pltpu
from jax.experimental.pallas import tpu_sc as plsc

VOCAB = 1000000
EMB = 64
BATCH = 4096
SEQ = 200

NUM_CORES = 2
NUM_SUBCORES = 16
NW = NUM_CORES * NUM_SUBCORES
SENT_PER_W = BATCH // NW
LANES = 16
NB = SENT_PER_W // LANES

VOCAB_PAD = 1000448
STAGE_CHUNK = 61440
STAGE_LAST_OFF = NUM_SUBCORES * STAGE_CHUNK
STAGE_LAST = VOCAB_PAD - STAGE_LAST_OFF

TC_BLK = 62464
MAIN_V = (VOCAB // TC_BLK) * TC_BLK
TAIL_V = VOCAB - MAIN_V


def _matvec_main_body(db_ref, v_ref, tabt_ref, out_ref):
    out_ref[...] = (
        jax.lax.dot_general(
            v_ref[...],
            tabt_ref[...],
            (((1,), (0,)), ((), ())),
            preferred_element_type=jnp.float32,
        )[0]
        + db_ref[0]
    )


def _matvec_tail_body(db_ref, v_ref, tab_ref, _prev_ref, out_ref):
    out_ref[pl.ds(0, TAIL_V)] = (
        jax.lax.dot_general(
            tab_ref[...],
            v_ref[...],
            (((1,), (1,)), ((), ())),
            preferred_element_type=jnp.float32,
        )[:, 0]
        + db_ref[0]
    )


def _tc_matvec(table, v1, db_arr):
    tabt = table.T
    main = pl.pallas_call(
        _matvec_main_body,
        grid=(MAIN_V // TC_BLK,),
        in_specs=[
            pl.BlockSpec(memory_space=pltpu.SMEM),
            pl.BlockSpec((1, EMB), lambda i: (0, 0)),
            pl.BlockSpec((EMB, TC_BLK), lambda i: (0, i)),
        ],
        out_specs=pl.BlockSpec((TC_BLK,), lambda i: (i,)),
        out_shape=jax.ShapeDtypeStruct((VOCAB_PAD,), jnp.float32),
    )(db_arr, v1, tabt)
    tail_rows = lax.slice(table, (MAIN_V, 0), (VOCAB, EMB))
    return pl.pallas_call(
        _matvec_tail_body,
        grid=(1,),
        in_specs=[
            pl.BlockSpec(memory_space=pltpu.SMEM),
            pl.BlockSpec((1, EMB), lambda i: (0, 0)),
            pl.BlockSpec((TAIL_V, EMB), lambda i: (0, 0)),
            pl.BlockSpec(memory_space=pl.ANY),
        ],
        out_specs=pl.BlockSpec((1024,), lambda i: (MAIN_V // 1024,)),
        out_shape=jax.ShapeDtypeStruct((VOCAB_PAD,), jnp.float32),
        input_output_aliases={3: 0},
    )(db_arr, v1, tail_rows, main)


def _sc_body(sent_hbm, tablev_hbm, out_hbm, idx_v, g_v, o_v, shared_v, sem):
    c = lax.axis_index("c")
    s = lax.axis_index("s")
    wid = s * NUM_CORES + c

    base = s * STAGE_CHUNK

    stage = pltpu.async_copy(
        tablev_hbm.at[pl.ds(base, STAGE_CHUNK)],
        shared_v.at[pl.ds(base, STAGE_CHUNK)],
        sem,
    )

    @pl.when(s == 0)
    def _stage_tail():
        pltpu.async_copy(
            tablev_hbm.at[pl.ds(STAGE_LAST_OFF, STAGE_LAST)],
            shared_v.at[pl.ds(STAGE_LAST_OFF, STAGE_LAST)],
            sem,
        ).wait()

    pltpu.sync_copy(sent_hbm.at[wid], idx_v)
    stage.wait()
    plsc.subcore_barrier()
    pltpu.async_copy(shared_v.at[idx_v], g_v, sem).wait()

    inv_l = jnp.float32(1.0 / SEQ)

    def body(r, accs):
        return tuple(
            a + g_v[pl.ds(r * SENT_PER_W + jb * LANES, LANES)]
            for jb, a in enumerate(accs)
        )

    accs = lax.fori_loop(
        0, SEQ, body, tuple(jnp.zeros((LANES,), jnp.float32) for _ in range(NB))
    )

    for jb in range(NB):
        d = accs[jb] * inv_l
        p0 = 1.0 / (1.0 + jnp.exp(-d))
        o_v[0, pl.ds(jb * LANES, LANES)] = p0
        o_v[1, pl.ds(jb * LANES, LANES)] = 1.0 - p0

    pltpu.sync_copy(o_v.at[0], out_hbm.at[0, wid])
    pltpu.sync_copy(o_v.at[1], out_hbm.at[1, wid])


_sc_kernel = functools.partial(
    pl.kernel,
    out_type=jax.ShapeDtypeStruct((2, NW, SENT_PER_W), jnp.float32),
    mesh=plsc.VectorSubcoreMesh(
        core_axis_name="c",
        subcore_axis_name="s",
        num_cores=NUM_CORES,
        num_subcores=NUM_SUBCORES,
    ),
    scratch_types=[
        pltpu.VMEM((SEQ * SENT_PER_W,), jnp.int32),
        pltpu.VMEM((SEQ * SENT_PER_W,), jnp.float32),
        pltpu.VMEM((2, SENT_PER_W), jnp.float32),
        pltpu.VMEM_SHARED((VOCAB_PAD,), jnp.float32),
        pltpu.SemaphoreType.DMA,
    ],
)(_sc_body)


def kernel(sentence, table, W2, b2):
    v1 = (W2[:, 0] - W2[:, 1]).reshape(1, EMB)
    db_arr = (b2[0] - b2[1]).reshape(1)
    tablev = _tc_matvec(table, v1, db_arr)
    sent_t = sentence.reshape(NW, SENT_PER_W, SEQ).transpose(0, 2, 1).reshape(
        NW, SEQ * SENT_PER_W
    )
    out2 = _sc_kernel(sent_t, tablev)
    return out2.reshape(2, BATCH).T
